# Initial kernel scaffold; baseline (speedup 1.0000x reference)
import jax
import jax.numpy as jnp
from jax.experimental import pallas as pl


def kernel(x, token_table, pos_table):
    raise NotImplementedError("write your pallas kernel here")


# speedup vs baseline: 2.7220x; 2.7220x over previous
import functools

import jax
import jax.numpy as jnp
from jax import lax
from jax.experimental import pallas as pl
from jax.experimental.pallas import tpu as pltpu
from jax.experimental.pallas import tpu_sc as plsc

B, S, D = 1024, 200, 64
NW = 32
ROWS_PER_W = (B * S) // NW
CHUNK = S
NCHUNK = ROWS_PER_W // CHUNK
HALF = CHUNK // 2
LANES = 16

_mesh = plsc.VectorSubcoreMesh(core_axis_name="c", subcore_axis_name="s")


@functools.partial(
    pl.kernel,
    mesh=_mesh,
    compiler_params=pltpu.CompilerParams(use_tc_tiling_on_sc=False),
    out_type=jax.ShapeDtypeStruct((B * S, D), jnp.float32),
    scratch_types=[
        pltpu.VMEM((NCHUNK, 2, HALF), jnp.int32),
        pltpu.VMEM((CHUNK, D), jnp.float32),
        pltpu.VMEM((S, D), jnp.float32),
        pltpu.SemaphoreType.DMA,
    ],
)
def _emb_kernel(x_hbm, tok_hbm, pos_hbm, out_hbm, idx_v, rows_v, pos_v, sem):
    wid = lax.axis_index("s") * 2 + lax.axis_index("c")
    base = wid * ROWS_PER_W
    pltpu.sync_copy(x_hbm.at[pl.ds(wid * NCHUNK, NCHUNK)], idx_v)
    pltpu.sync_copy(pos_hbm, pos_v)

    def chunk_body(ci, carry):
        g0 = pltpu.async_copy(
            tok_hbm.at[idx_v.at[ci, 0]], rows_v.at[pl.ds(0, HALF)], sem
        )
        g1 = pltpu.async_copy(
            tok_hbm.at[idx_v.at[ci, 1]], rows_v.at[pl.ds(HALF, HALF)], sem
        )
        g0.wait()
        g1.wait()

        def row_body(r, c2):
            for c in range(D // LANES):
                plsc.addupdate(
                    rows_v.at[r, pl.ds(c * LANES, LANES)],
                    pos_v[r, pl.ds(c * LANES, LANES)],
                )
            return c2

        lax.fori_loop(0, CHUNK, row_body, 0)
        pltpu.sync_copy(rows_v, out_hbm.at[pl.ds(base + ci * CHUNK, CHUNK)])
        return carry

    lax.fori_loop(0, NCHUNK, chunk_body, 0)


def kernel(x, token_table, pos_table):
    x3 = x.reshape(B, 2, HALF).astype(jnp.int32)
    flat = _emb_kernel(x3, token_table, pos_table)
    return flat.reshape(B, S, D)


# speedup vs baseline: 2.7674x; 1.0167x over previous
import functools

import jax
import jax.numpy as jnp
from jax import lax
from jax.experimental import pallas as pl
from jax.experimental.pallas import tpu as pltpu
from jax.experimental.pallas import tpu_sc as plsc

B, S, D = 1024, 200, 64
NW = 32
NSUB = 16
ROWS_PER_W = (B * S) // NW
CHUNK = S
NCHUNK = ROWS_PER_W // CHUNK
HALF = CHUNK // 2

_mesh = plsc.VectorSubcoreMesh(core_axis_name="c", subcore_axis_name="s")


@functools.partial(
    pl.kernel,
    mesh=_mesh,
    compiler_params=pltpu.CompilerParams(use_tc_tiling_on_sc=False),
    out_type=jax.ShapeDtypeStruct((B * S, D), jnp.float32),
    scratch_types=[
        pltpu.VMEM((NCHUNK, 2, HALF), jnp.int32),
        pltpu.VMEM((CHUNK, D), jnp.float32),
        pltpu.VMEM_SHARED((NSUB * CHUNK, D), jnp.float32),
        pltpu.VMEM((S, D), jnp.float32),
        pltpu.VMEM((2, HALF), jnp.int32),
        pltpu.SemaphoreType.DMA,
        pltpu.SemaphoreType.DMA,
    ],
)
def _emb_kernel(x_hbm, tok_hbm, pos_hbm, iota_hbm, out_hbm,
                idx_v, rows_v, acc_sh, pos_v, iota_v, sem, sem2):
    sub = lax.axis_index("s")
    wid = sub * 2 + lax.axis_index("c")
    base = wid * ROWS_PER_W
    region = sub * CHUNK
    pltpu.sync_copy(x_hbm.at[pl.ds(wid * NCHUNK, NCHUNK)], idx_v)
    pltpu.sync_copy(iota_hbm.at[sub], iota_v)
    pltpu.sync_copy(pos_hbm, pos_v)

    def chunk_body(ci, carry):
        refill = pltpu.async_copy(
            pos_v, acc_sh.at[pl.ds(region, CHUNK)], sem2
        )
        g0 = pltpu.async_copy(
            tok_hbm.at[idx_v.at[ci, 0]], rows_v.at[pl.ds(0, HALF)], sem
        )
        g1 = pltpu.async_copy(
            tok_hbm.at[idx_v.at[ci, 1]], rows_v.at[pl.ds(HALF, HALF)], sem
        )
        refill.wait()
        g0.wait()
        g1.wait()
        pltpu.sync_copy(
            rows_v.at[pl.ds(0, HALF)], acc_sh.at[iota_v.at[0]], add=True
        )
        pltpu.sync_copy(
            rows_v.at[pl.ds(HALF, HALF)], acc_sh.at[iota_v.at[1]], add=True
        )
        pltpu.sync_copy(
            acc_sh.at[pl.ds(region, CHUNK)],
            out_hbm.at[pl.ds(base + ci * CHUNK, CHUNK)],
        )
        return carry

    lax.fori_loop(0, NCHUNK, chunk_body, 0)


def kernel(x, token_table, pos_table):
    x3 = x.reshape(B, 2, HALF).astype(jnp.int32)
    iota = (
        jnp.arange(S, dtype=jnp.int32)[None, :]
        + (CHUNK * jnp.arange(NSUB, dtype=jnp.int32))[:, None]
    ).reshape(NSUB, 2, HALF)
    flat = _emb_kernel(x3, token_table, pos_table, iota)
    return flat.reshape(B, S, D)


# speedup vs baseline: 2.9553x; 1.0679x over previous
import functools

import jax
import jax.numpy as jnp
from jax import lax
from jax.experimental import pallas as pl
from jax.experimental.pallas import tpu as pltpu
from jax.experimental.pallas import tpu_sc as plsc

B, S, D = 1024, 200, 64
NW = 32
NSUB = 16
ROWS_PER_W = (B * S) // NW
CHUNK = S
NCHUNK = ROWS_PER_W // CHUNK
HALF = CHUNK // 2
NBUF = 2

_mesh = plsc.VectorSubcoreMesh(core_axis_name="c", subcore_axis_name="s")


@functools.partial(
    pl.kernel,
    mesh=_mesh,
    compiler_params=pltpu.CompilerParams(use_tc_tiling_on_sc=False),
    out_type=jax.ShapeDtypeStruct((B * S, D), jnp.float32),
    scratch_types=[
        pltpu.VMEM((NCHUNK, 2, HALF), jnp.int32),
        pltpu.VMEM((NBUF, CHUNK, D), jnp.float32),
        pltpu.VMEM_SHARED((NSUB * NBUF * CHUNK, D), jnp.float32),
        pltpu.VMEM((S, D), jnp.float32),
        pltpu.VMEM((NBUF, 2, HALF), jnp.int32),
        pltpu.SemaphoreType.DMA,
        pltpu.SemaphoreType.DMA,
        pltpu.SemaphoreType.DMA,
        pltpu.SemaphoreType.DMA,
    ],
)
def _emb_kernel(x_hbm, tok_hbm, pos_hbm, iota_hbm, out_hbm,
                idx_v, rows_v, acc_sh, pos_v, iota_v,
                semg0, semg1, semw0, semw1):
    sub = lax.axis_index("s")
    wid = sub * 2 + lax.axis_index("c")
    base = wid * ROWS_PER_W
    pltpu.sync_copy(x_hbm.at[pl.ds(wid * NCHUNK, NCHUNK)], idx_v)
    pltpu.sync_copy(iota_hbm.at[sub], iota_v)
    pltpu.sync_copy(pos_hbm, pos_v)

    semg = [semg0, semg1]
    semw = [semw0, semw1]

    def gather(ci, b):
        return [
            pltpu.async_copy(
                tok_hbm.at[idx_v.at[ci, h]],
                rows_v.at[b, pl.ds(h * HALF, HALF)],
                semg[b],
            )
            for h in (0, 1)
        ]

    g = {0: gather(0, 0), 1: gather(1, 1)}
    for ci in range(NCHUNK):
        b = ci % NBUF
        region = (sub * NBUF + b) * CHUNK
        pltpu.sync_copy(pos_v, acc_sh.at[pl.ds(region, CHUNK)])
        for hnd in g[ci]:
            hnd.wait()
        pltpu.sync_copy(
            rows_v.at[b, pl.ds(0, HALF)], acc_sh.at[iota_v.at[b, 0]],
            add=True,
        )
        pltpu.sync_copy(
            rows_v.at[b, pl.ds(HALF, HALF)], acc_sh.at[iota_v.at[b, 1]],
            add=True,
        )
        if ci + NBUF < NCHUNK:
            g[ci + NBUF] = gather(ci + NBUF, b)
        pltpu.sync_copy(
            acc_sh.at[pl.ds(region, CHUNK)],
            out_hbm.at[pl.ds(base + ci * CHUNK, CHUNK)],
        )


def kernel(x, token_table, pos_table):
    x3 = x.reshape(B, 2, HALF).astype(jnp.int32)
    iota = (
        jnp.arange(S, dtype=jnp.int32)[None, None, :]
        + (CHUNK * jnp.arange(NBUF, dtype=jnp.int32))[None, :, None]
        + (NBUF * CHUNK * jnp.arange(NSUB, dtype=jnp.int32))[:, None, None]
    ).reshape(NSUB, NBUF, 2, HALF)
    flat = _emb_kernel(x3, token_table, pos_table, iota)
    return flat.reshape(B, S, D)
